# baseline (speedup 1.0000x reference)
import jax
import jax.numpy as jnp
from jax import lax
from jax.experimental import pallas as pl
from jax.experimental.pallas import tpu as pltpu
from jax.experimental.pallas import tpu_sc as plsc

N = 50000
M = 16
ORIG = 128
NBR = 16
AF = 64
NC = 3
HF = 128
N0 = 1000
APC = 50
EPS = 1e-5

NWORK = 32
TW = 128


def _make_sc_gather(n_ops, group, d):
    assert n_ops % NWORK == 0
    ops_w = n_ops // NWORK
    assert ops_w % group == 0 and ops_w % 8 == 0
    n_groups = ops_w // group
    rows_g = group * 128

    def body(table_hbm, idx_hbm, out_hbm, idx_v, rows_v, sem):
        wid = lax.axis_index("s") * 2 + lax.axis_index("c")
        op_base = wid * ops_w
        pltpu.sync_copy(idx_hbm.at[pl.ds(op_base, ops_w)], idx_v)

        def grp(g, carry):
            cps = []
            for j in range(group):
                cps.append(
                    pltpu.async_copy(
                        table_hbm.at[idx_v.at[g * group + j]],
                        rows_v.at[pl.ds(j * 128, 128)],
                        sem,
                    )
                )
            for c in cps:
                c.wait()
            row0 = (op_base + g * group) * 128
            pltpu.sync_copy(rows_v, out_hbm.at[pl.ds(row0, rows_g)])
            return carry

        lax.fori_loop(0, n_groups, grp, 0)

    def run(table, idx2d):
        mesh = plsc.VectorSubcoreMesh(core_axis_name="c", subcore_axis_name="s")
        k = pl.kernel(
            body,
            mesh=mesh,
            out_type=jax.ShapeDtypeStruct((n_ops * 128, d), jnp.float32),
            scratch_types=[
                pltpu.VMEM((ops_w, 128), jnp.int32),
                pltpu.VMEM((rows_g, d), jnp.float32),
                pltpu.SemaphoreType.DMA,
            ],
        )
        return k(table, idx2d)

    return run


_NB_OPS = 6400
_NB_PAD = _NB_OPS * 128
_gather_nbr = _make_sc_gather(_NB_OPS, 4, TW)

_PL_OPS = 512
_PL_PAD = _PL_OPS * 128
_gather_pool = _make_sc_gather(_PL_OPS, 4, TW)


_BN_ROWS = 400
_NBLK = N // _BN_ROWS


def _embed_k(a_ref, wt_ref, b_ref, o_ref):
    o_ref[...] = (
        jnp.dot(a_ref[...], wt_ref[...], preferred_element_type=jnp.float32, precision=lax.Precision.HIGHEST)
        + b_ref[...]
    )


def _pass1_k(x_ref, g_ref, e_ref, wst, wnt, wet, b_ref, acc_out, acc):
    i = pl.program_id(0)
    p = jnp.dot(x_ref[...], wst[...], preferred_element_type=jnp.float32, precision=lax.Precision.HIGHEST) + b_ref[...]
    gf = jnp.dot(g_ref[...], wnt[...], preferred_element_type=jnp.float32, precision=lax.Precision.HIGHEST)
    gf = gf + jnp.dot(e_ref[...], wet[...], preferred_element_type=jnp.float32, precision=lax.Precision.HIGHEST)
    g3 = gf.reshape(_BN_ROWS, M, 2 * AF) + p[:, None, :]
    s = jnp.sum(g3, axis=(0, 1))[None, :]
    s2 = jnp.sum(g3 * g3, axis=(0, 1))[None, :]

    @pl.when(i == 0)
    def _():
        acc[...] = jnp.zeros_like(acc)

    acc[...] += jnp.concatenate([s, s2], axis=0)

    @pl.when(i == _NBLK - 1)
    def _():
        acc_out[...] = acc[...]


def _pass2_k(
    x_ref, g_ref, e_ref,
    wstF, wntF, wetF, bF,
    wstC, wntC, wetC, bC,
    ns_out, acc_out, acc,
):
    i = pl.program_id(0)
    xb = x_ref[...]
    gb = g_ref[...]
    eb = e_ref[...]
    pF = jnp.dot(xb, wstF[...], preferred_element_type=jnp.float32, precision=lax.Precision.HIGHEST) + bF[...]
    rF = jnp.dot(gb, wntF[...], preferred_element_type=jnp.float32, precision=lax.Precision.HIGHEST)
    rF = rF + jnp.dot(eb, wetF[...], preferred_element_type=jnp.float32, precision=lax.Precision.HIGHEST)
    g3F = rF.reshape(_BN_ROWS, M, AF) + pF[:, None, :]
    pC = jnp.dot(xb, wstC[...], preferred_element_type=jnp.float32, precision=lax.Precision.HIGHEST) + bC[...]
    rC = jnp.dot(gb, wntC[...], preferred_element_type=jnp.float32, precision=lax.Precision.HIGHEST)
    rC = rC + jnp.dot(eb, wetC[...], preferred_element_type=jnp.float32, precision=lax.Precision.HIGHEST)
    g3C = rC.reshape(_BN_ROWS, M, AF) + pC[:, None, :]

    filt = jax.nn.sigmoid(g3F)
    core = jax.nn.softplus(g3C)
    ns = jnp.sum(filt * core, axis=1)
    ns_out[...] = ns

    s = jnp.sum(ns, axis=0)[None, :]
    s2 = jnp.sum(ns * ns, axis=0)[None, :]

    @pl.when(i == 0)
    def _():
        acc[...] = jnp.zeros_like(acc)

    acc[...] += jnp.concatenate([s, s2], axis=0)

    @pl.when(i == _NBLK - 1)
    def _():
        acc_out[...] = acc[...]


def _pass3_k(x_ref, ns_ref, s_ref, t_ref, o_ref):
    v = jax.nn.softplus(
        x_ref[:, :AF] + ns_ref[...] * s_ref[...] + t_ref[...]
    )
    o_ref[...] = jnp.concatenate(
        [v, jnp.zeros((_BN_ROWS, TW - AF), jnp.float32)], axis=1
    )


_BNC = 200
_NBLKC = N0 // _BNC


def _final_k(p_ref, w1t, b1_, wot, bo_, out_ref, crys_ref):
    pr = p_ref[:, :AF]
    cf = jnp.mean(pr.reshape(_BNC, APC, AF), axis=1)
    crys_ref[...] = cf
    h = (
        jnp.dot(jax.nn.softplus(cf), w1t[...], preferred_element_type=jnp.float32, precision=lax.Precision.HIGHEST)
        + b1_[...]
    )
    h = jax.nn.softplus(h)
    out_ref[...] = (
        jnp.dot(h, wot[...], preferred_element_type=jnp.float32, precision=lax.Precision.HIGHEST) + bo_[...]
    )


def _tc_params():
    return pltpu.CompilerParams(dimension_semantics=("arbitrary",))


def _embed(atom_fea, W_embed, b_embed):
    wt = jnp.concatenate(
        [W_embed.T, jnp.zeros((ORIG, TW - AF), jnp.float32)], axis=1
    )
    bp = jnp.concatenate([b_embed, jnp.zeros((TW - AF,), jnp.float32)])[None, :]
    return pl.pallas_call(
        _embed_k,
        grid=(_NBLK,),
        in_specs=[
            pl.BlockSpec((_BN_ROWS, ORIG), lambda i: (i, 0)),
            pl.BlockSpec((ORIG, TW), lambda i: (0, 0)),
            pl.BlockSpec((1, TW), lambda i: (0, 0)),
        ],
        out_specs=pl.BlockSpec((_BN_ROWS, TW), lambda i: (i, 0)),
        out_shape=jax.ShapeDtypeStruct((N, TW), jnp.float32),
        compiler_params=_tc_params(),
    )(atom_fea, wt, bp)


def _pass1(x, gpad, eflat, wst, wnt, wet, bias):
    return pl.pallas_call(
        _pass1_k,
        grid=(_NBLK,),
        in_specs=[
            pl.BlockSpec((_BN_ROWS, TW), lambda i: (i, 0)),
            pl.BlockSpec((_BN_ROWS * M, TW), lambda i: (i, 0)),
            pl.BlockSpec((_BN_ROWS * M, NBR), lambda i: (i, 0)),
            pl.BlockSpec((TW, 2 * AF), lambda i: (0, 0)),
            pl.BlockSpec((TW, 2 * AF), lambda i: (0, 0)),
            pl.BlockSpec((NBR, 2 * AF), lambda i: (0, 0)),
            pl.BlockSpec((1, 2 * AF), lambda i: (0, 0)),
        ],
        out_specs=pl.BlockSpec((2, 2 * AF), lambda i: (0, 0)),
        out_shape=jax.ShapeDtypeStruct((2, 2 * AF), jnp.float32),
        scratch_shapes=[pltpu.VMEM((2, 2 * AF), jnp.float32)],
        compiler_params=_tc_params(),
    )(x, gpad, eflat, wst, wnt, wet, bias)


def _pass2(x, gpad, eflat, wF, wC):
    return pl.pallas_call(
        _pass2_k,
        grid=(_NBLK,),
        in_specs=[
            pl.BlockSpec((_BN_ROWS, TW), lambda i: (i, 0)),
            pl.BlockSpec((_BN_ROWS * M, TW), lambda i: (i, 0)),
            pl.BlockSpec((_BN_ROWS * M, NBR), lambda i: (i, 0)),
            pl.BlockSpec((TW, AF), lambda i: (0, 0)),
            pl.BlockSpec((TW, AF), lambda i: (0, 0)),
            pl.BlockSpec((NBR, AF), lambda i: (0, 0)),
            pl.BlockSpec((1, AF), lambda i: (0, 0)),
            pl.BlockSpec((TW, AF), lambda i: (0, 0)),
            pl.BlockSpec((TW, AF), lambda i: (0, 0)),
            pl.BlockSpec((NBR, AF), lambda i: (0, 0)),
            pl.BlockSpec((1, AF), lambda i: (0, 0)),
        ],
        out_specs=[
            pl.BlockSpec((_BN_ROWS, AF), lambda i: (i, 0)),
            pl.BlockSpec((2, AF), lambda i: (0, 0)),
        ],
        out_shape=[
            jax.ShapeDtypeStruct((N, AF), jnp.float32),
            jax.ShapeDtypeStruct((2, AF), jnp.float32),
        ],
        scratch_shapes=[pltpu.VMEM((2, AF), jnp.float32)],
        compiler_params=_tc_params(),
    )(x, gpad, eflat, *wF, *wC)


def _pass3(x, ns, s2, t2):
    return pl.pallas_call(
        _pass3_k,
        grid=(_NBLK,),
        in_specs=[
            pl.BlockSpec((_BN_ROWS, TW), lambda i: (i, 0)),
            pl.BlockSpec((_BN_ROWS, AF), lambda i: (i, 0)),
            pl.BlockSpec((1, AF), lambda i: (0, 0)),
            pl.BlockSpec((1, AF), lambda i: (0, 0)),
        ],
        out_specs=pl.BlockSpec((_BN_ROWS, TW), lambda i: (i, 0)),
        out_shape=jax.ShapeDtypeStruct((N, TW), jnp.float32),
        compiler_params=_tc_params(),
    )(x, ns, s2, t2)


def _final(pooled, W1, b1, Wo, bo):
    return pl.pallas_call(
        _final_k,
        grid=(_NBLKC,),
        in_specs=[
            pl.BlockSpec((_BNC * APC, TW), lambda i: (i, 0)),
            pl.BlockSpec((AF, HF), lambda i: (0, 0)),
            pl.BlockSpec((1, HF), lambda i: (0, 0)),
            pl.BlockSpec((HF, 1), lambda i: (0, 0)),
            pl.BlockSpec((1, 1), lambda i: (0, 0)),
        ],
        out_specs=[
            pl.BlockSpec((_BNC, 1), lambda i: (i, 0)),
            pl.BlockSpec((_BNC, AF), lambda i: (i, 0)),
        ],
        out_shape=[
            jax.ShapeDtypeStruct((N0, 1), jnp.float32),
            jax.ShapeDtypeStruct((N0, AF), jnp.float32),
        ],
        compiler_params=_tc_params(),
    )(pooled, W1.T, b1[None, :], Wo.T, bo[None, :])


def _bn_affine(acc, g, b, rows):
    mean = acc[0] / rows
    var = acc[1] / rows - mean * mean
    s = g * lax.rsqrt(var + EPS)
    t = b - mean * s
    return s, t


def _pad_rows(w):
    return jnp.concatenate([w, jnp.zeros((TW - AF, w.shape[1]), w.dtype)], axis=0)


def kernel(atom_fea, nbr_fea, nbr_fea_idx, crystal_atom_idx, W_embed, b_embed,
           convW, convB, conv_g1, conv_b1, conv_g2, conv_b2, W1, b1, Wo, bo):
    eflat = nbr_fea.reshape(N * M, NBR)
    nb_idx = jnp.concatenate(
        [nbr_fea_idx.reshape(-1), jnp.zeros((_NB_PAD - N * M,), jnp.int32)]
    ).reshape(_NB_OPS, 128)
    pool_idx = jnp.concatenate(
        [crystal_atom_idx.reshape(-1), jnp.zeros((_PL_PAD - N0 * APC,), jnp.int32)]
    ).reshape(_PL_OPS, 128)

    x = _embed(atom_fea, W_embed, b_embed)

    for i in range(NC):
        W = convW[i]
        bvec = convB[i]
        wst = _pad_rows(W[:, :AF].T)
        wnt = _pad_rows(W[:, AF:2 * AF].T)
        wet = W[:, 2 * AF:].T

        gpad = _gather_nbr(x, nb_idx)
        acc1 = _pass1(x, gpad, eflat, wst, wnt, wet, bvec[None, :])
        s1, t1 = _bn_affine(acc1, conv_g1[i], conv_b1[i], float(N * M))

        wstS = wst * s1[None, :]
        wntS = wnt * s1[None, :]
        wetS = wet * s1[None, :]
        bS = bvec * s1 + t1
        wF = (wstS[:, :AF], wntS[:, :AF], wetS[:, :AF], bS[None, :AF])
        wC = (wstS[:, AF:], wntS[:, AF:], wetS[:, AF:], bS[None, AF:])

        ns, acc2 = _pass2(x, gpad, eflat, wF, wC)
        s2, t2 = _bn_affine(acc2, conv_g2[i], conv_b2[i], float(N))
        x = _pass3(x, ns, s2[None, :], t2[None, :])

    pooled = _gather_pool(x, pool_idx)
    out, crys_fea = _final(pooled, W1, b1, Wo, bo)
    return (out, crys_fea)


# speedup vs baseline: 1.7632x; 1.7632x over previous
import jax
import jax.numpy as jnp
from jax import lax
from jax.experimental import pallas as pl
from jax.experimental.pallas import tpu as pltpu
from jax.experimental.pallas import tpu_sc as plsc

N = 50000
M = 16
ORIG = 128
NBR = 16
AF = 64
NC = 3
HF = 128
N0 = 1000
APC = 50
EPS = 1e-5

NWORK = 32
TW = 128


def _make_sc_gather(n_ops, group, d):
    assert n_ops % NWORK == 0
    ops_w = n_ops // NWORK
    assert ops_w % (2 * group) == 0 and ops_w % 8 == 0
    n_ss = ops_w // (2 * group)
    rows_g = group * 128

    def body(table_hbm, idx_hbm, out_hbm, idx_v, rows0, rows1, a0, a1, b0, b1):
        wid = lax.axis_index("s") * 2 + lax.axis_index("c")
        op_base = wid * ops_w
        pltpu.sync_copy(idx_hbm.at[pl.ds(op_base, ops_w)], idx_v)
        bufs = (rows0, rows1)
        gsems = (a0, a1)
        wsems = (b0, b1)

        def wb_copy(buf, g, sem):
            row0 = (op_base + g * group) * 128
            return pltpu.make_async_copy(
                buf, out_hbm.at[pl.ds(row0, rows_g)], sem
            )

        def sstep(ss, carry):
            for t in range(2):
                g = ss * 2 + t

                @pl.when(ss > 0)
                def _():
                    wb_copy(bufs[t], g - 2, wsems[t]).wait()

                for j in range(group):
                    pltpu.async_copy(
                        table_hbm.at[idx_v.at[g * group + j]],
                        bufs[t].at[pl.ds(j * 128, 128)],
                        gsems[t],
                    )
            for t in range(2):
                g = ss * 2 + t
                for j in range(group):
                    pltpu.make_async_copy(
                        table_hbm.at[idx_v.at[g * group + j]],
                        bufs[t].at[pl.ds(j * 128, 128)],
                        gsems[t],
                    ).wait()
                wb_copy(bufs[t], g, wsems[t]).start()
            return carry

        lax.fori_loop(0, n_ss, sstep, 0)
        for t in range(2):
            wb_copy(bufs[t], (n_ss - 1) * 2 + t, wsems[t]).wait()

    def run(table, idx2d):
        mesh = plsc.VectorSubcoreMesh(core_axis_name="c", subcore_axis_name="s")
        k = pl.kernel(
            body,
            mesh=mesh,
            out_type=jax.ShapeDtypeStruct((n_ops * 128, d), jnp.float32),
            scratch_types=[
                pltpu.VMEM((ops_w, 128), jnp.int32),
                pltpu.VMEM((rows_g, d), jnp.float32),
                pltpu.VMEM((rows_g, d), jnp.float32),
                pltpu.SemaphoreType.DMA,
                pltpu.SemaphoreType.DMA,
                pltpu.SemaphoreType.DMA,
                pltpu.SemaphoreType.DMA,
            ],
        )
        return k(table, idx2d)

    return run


_NB_OPS = 6400
_NB_PAD = _NB_OPS * 128
_gather_nbr = _make_sc_gather(_NB_OPS, 2, TW)

_PL_OPS = 512
_PL_PAD = _PL_OPS * 128
_gather_pool = _make_sc_gather(_PL_OPS, 2, TW)


_BN_ROWS = 400
_NBLK = N // _BN_ROWS


def _embed_k(a_ref, wt_ref, b_ref, o_ref):
    o_ref[...] = (
        jnp.dot(a_ref[...], wt_ref[...], preferred_element_type=jnp.float32, precision=lax.Precision.HIGHEST)
        + b_ref[...]
    )


def _pass1_k(x_ref, g_ref, e_ref, wst, wnt, wet, b_ref, acc_out, acc):
    i = pl.program_id(0)
    p = jnp.dot(x_ref[...], wst[...], preferred_element_type=jnp.float32, precision=lax.Precision.HIGHEST) + b_ref[...]
    gf = jnp.dot(g_ref[...].astype(jnp.bfloat16), wnt[...],
                 preferred_element_type=jnp.float32)
    gf = gf + jnp.dot(e_ref[...], wet[...], preferred_element_type=jnp.float32)
    g3 = gf.reshape(_BN_ROWS, M, 2 * AF) + p[:, None, :]
    s = jnp.sum(g3, axis=(0, 1))[None, :]
    s2 = jnp.sum(g3 * g3, axis=(0, 1))[None, :]

    @pl.when(i == 0)
    def _():
        acc[...] = jnp.zeros_like(acc)

    acc[...] += jnp.concatenate([s, s2], axis=0)

    @pl.when(i == _NBLK - 1)
    def _():
        acc_out[...] = acc[...]


def _pass2_k(
    x_ref, g_ref, e_ref,
    wstF, wntF, wetF, bF,
    wstC, wntC, wetC, bC,
    ns_out, acc_out, acc,
):
    i = pl.program_id(0)
    xb = x_ref[...]
    gb = g_ref[...]
    eb = e_ref[...]
    gbf = gb.astype(jnp.bfloat16)
    pF = jnp.dot(xb, wstF[...], preferred_element_type=jnp.float32, precision=lax.Precision.HIGHEST) + bF[...]
    rF = jnp.dot(gbf, wntF[...], preferred_element_type=jnp.float32)
    rF = rF + jnp.dot(eb, wetF[...], preferred_element_type=jnp.float32)
    g3F = rF.reshape(_BN_ROWS, M, AF) + pF[:, None, :]
    pC = jnp.dot(xb, wstC[...], preferred_element_type=jnp.float32, precision=lax.Precision.HIGHEST) + bC[...]
    rC = jnp.dot(gbf, wntC[...], preferred_element_type=jnp.float32)
    rC = rC + jnp.dot(eb, wetC[...], preferred_element_type=jnp.float32)
    g3C = rC.reshape(_BN_ROWS, M, AF) + pC[:, None, :]

    filt = jax.nn.sigmoid(g3F)
    core = jax.nn.softplus(g3C)
    ns = jnp.sum(filt * core, axis=1)
    ns_out[...] = ns

    s = jnp.sum(ns, axis=0)[None, :]
    s2 = jnp.sum(ns * ns, axis=0)[None, :]

    @pl.when(i == 0)
    def _():
        acc[...] = jnp.zeros_like(acc)

    acc[...] += jnp.concatenate([s, s2], axis=0)

    @pl.when(i == _NBLK - 1)
    def _():
        acc_out[...] = acc[...]


def _pass3_k(x_ref, ns_ref, s_ref, t_ref, o_ref):
    v = jax.nn.softplus(
        x_ref[:, :AF] + ns_ref[...] * s_ref[...] + t_ref[...]
    )
    o_ref[...] = jnp.concatenate(
        [v, jnp.zeros((_BN_ROWS, TW - AF), jnp.float32)], axis=1
    )


_BNC = 200
_NBLKC = N0 // _BNC


def _final_k(p_ref, w1t, b1_, wot, bo_, out_ref, crys_ref):
    pr = p_ref[:, :AF]
    cf = jnp.mean(pr.reshape(_BNC, APC, AF), axis=1)
    crys_ref[...] = cf
    h = (
        jnp.dot(jax.nn.softplus(cf), w1t[...], preferred_element_type=jnp.float32, precision=lax.Precision.HIGHEST)
        + b1_[...]
    )
    h = jax.nn.softplus(h)
    out_ref[...] = (
        jnp.dot(h, wot[...], preferred_element_type=jnp.float32, precision=lax.Precision.HIGHEST) + bo_[...]
    )


def _tc_params():
    return pltpu.CompilerParams(dimension_semantics=("arbitrary",))


def _embed(atom_fea, W_embed, b_embed):
    wt = jnp.concatenate(
        [W_embed.T, jnp.zeros((ORIG, TW - AF), jnp.float32)], axis=1
    )
    bp = jnp.concatenate([b_embed, jnp.zeros((TW - AF,), jnp.float32)])[None, :]
    return pl.pallas_call(
        _embed_k,
        grid=(_NBLK,),
        in_specs=[
            pl.BlockSpec((_BN_ROWS, ORIG), lambda i: (i, 0)),
            pl.BlockSpec((ORIG, TW), lambda i: (0, 0)),
            pl.BlockSpec((1, TW), lambda i: (0, 0)),
        ],
        out_specs=pl.BlockSpec((_BN_ROWS, TW), lambda i: (i, 0)),
        out_shape=jax.ShapeDtypeStruct((N, TW), jnp.float32),
        compiler_params=_tc_params(),
    )(atom_fea, wt, bp)


def _pass1(x, gpad, eflat, wst, wnt, wet, bias):
    return pl.pallas_call(
        _pass1_k,
        grid=(_NBLK,),
        in_specs=[
            pl.BlockSpec((_BN_ROWS, TW), lambda i: (i, 0)),
            pl.BlockSpec((_BN_ROWS * M, TW), lambda i: (i, 0)),
            pl.BlockSpec((_BN_ROWS * M, NBR), lambda i: (i, 0)),
            pl.BlockSpec((TW, 2 * AF), lambda i: (0, 0)),
            pl.BlockSpec((TW, 2 * AF), lambda i: (0, 0)),
            pl.BlockSpec((NBR, 2 * AF), lambda i: (0, 0)),
            pl.BlockSpec((1, 2 * AF), lambda i: (0, 0)),
        ],
        out_specs=pl.BlockSpec((2, 2 * AF), lambda i: (0, 0)),
        out_shape=jax.ShapeDtypeStruct((2, 2 * AF), jnp.float32),
        scratch_shapes=[pltpu.VMEM((2, 2 * AF), jnp.float32)],
        compiler_params=_tc_params(),
    )(x, gpad, eflat, wst, wnt, wet, bias)


def _pass2(x, gpad, eflat, wF, wC):
    return pl.pallas_call(
        _pass2_k,
        grid=(_NBLK,),
        in_specs=[
            pl.BlockSpec((_BN_ROWS, TW), lambda i: (i, 0)),
            pl.BlockSpec((_BN_ROWS * M, TW), lambda i: (i, 0)),
            pl.BlockSpec((_BN_ROWS * M, NBR), lambda i: (i, 0)),
            pl.BlockSpec((TW, AF), lambda i: (0, 0)),
            pl.BlockSpec((TW, AF), lambda i: (0, 0)),
            pl.BlockSpec((NBR, AF), lambda i: (0, 0)),
            pl.BlockSpec((1, AF), lambda i: (0, 0)),
            pl.BlockSpec((TW, AF), lambda i: (0, 0)),
            pl.BlockSpec((TW, AF), lambda i: (0, 0)),
            pl.BlockSpec((NBR, AF), lambda i: (0, 0)),
            pl.BlockSpec((1, AF), lambda i: (0, 0)),
        ],
        out_specs=[
            pl.BlockSpec((_BN_ROWS, AF), lambda i: (i, 0)),
            pl.BlockSpec((2, AF), lambda i: (0, 0)),
        ],
        out_shape=[
            jax.ShapeDtypeStruct((N, AF), jnp.float32),
            jax.ShapeDtypeStruct((2, AF), jnp.float32),
        ],
        scratch_shapes=[pltpu.VMEM((2, AF), jnp.float32)],
        compiler_params=_tc_params(),
    )(x, gpad, eflat, *wF, *wC)


def _pass3(x, ns, s2, t2):
    return pl.pallas_call(
        _pass3_k,
        grid=(_NBLK,),
        in_specs=[
            pl.BlockSpec((_BN_ROWS, TW), lambda i: (i, 0)),
            pl.BlockSpec((_BN_ROWS, AF), lambda i: (i, 0)),
            pl.BlockSpec((1, AF), lambda i: (0, 0)),
            pl.BlockSpec((1, AF), lambda i: (0, 0)),
        ],
        out_specs=pl.BlockSpec((_BN_ROWS, TW), lambda i: (i, 0)),
        out_shape=jax.ShapeDtypeStruct((N, TW), jnp.float32),
        compiler_params=_tc_params(),
    )(x, ns, s2, t2)


def _final(pooled, W1, b1, Wo, bo):
    return pl.pallas_call(
        _final_k,
        grid=(_NBLKC,),
        in_specs=[
            pl.BlockSpec((_BNC * APC, TW), lambda i: (i, 0)),
            pl.BlockSpec((AF, HF), lambda i: (0, 0)),
            pl.BlockSpec((1, HF), lambda i: (0, 0)),
            pl.BlockSpec((HF, 1), lambda i: (0, 0)),
            pl.BlockSpec((1, 1), lambda i: (0, 0)),
        ],
        out_specs=[
            pl.BlockSpec((_BNC, 1), lambda i: (i, 0)),
            pl.BlockSpec((_BNC, AF), lambda i: (i, 0)),
        ],
        out_shape=[
            jax.ShapeDtypeStruct((N0, 1), jnp.float32),
            jax.ShapeDtypeStruct((N0, AF), jnp.float32),
        ],
        compiler_params=_tc_params(),
    )(pooled, W1.T, b1[None, :], Wo.T, bo[None, :])


def _bn_affine(acc, g, b, rows):
    mean = acc[0] / rows
    var = acc[1] / rows - mean * mean
    s = g * lax.rsqrt(var + EPS)
    t = b - mean * s
    return s, t


def _pad_rows(w):
    return jnp.concatenate([w, jnp.zeros((TW - AF, w.shape[1]), w.dtype)], axis=0)


def kernel(atom_fea, nbr_fea, nbr_fea_idx, crystal_atom_idx, W_embed, b_embed,
           convW, convB, conv_g1, conv_b1, conv_g2, conv_b2, W1, b1, Wo, bo):
    eflat = nbr_fea.reshape(N * M, NBR).astype(jnp.bfloat16)
    nb_idx = jnp.concatenate(
        [nbr_fea_idx.reshape(-1), jnp.zeros((_NB_PAD - N * M,), jnp.int32)]
    ).reshape(_NB_OPS, 128)
    pool_idx = jnp.concatenate(
        [crystal_atom_idx.reshape(-1), jnp.zeros((_PL_PAD - N0 * APC,), jnp.int32)]
    ).reshape(_PL_OPS, 128)

    x = _embed(atom_fea, W_embed, b_embed)

    for i in range(NC):
        W = convW[i]
        bvec = convB[i]
        wst = _pad_rows(W[:, :AF].T)
        wnt = _pad_rows(W[:, AF:2 * AF].T)
        wet = W[:, 2 * AF:].T
        wnt_b = wnt.astype(jnp.bfloat16)
        wet_b = wet.astype(jnp.bfloat16)

        gpad = _gather_nbr(x, nb_idx)
        acc1 = _pass1(x, gpad, eflat, wst, wnt_b, wet_b, bvec[None, :])
        s1, t1 = _bn_affine(acc1, conv_g1[i], conv_b1[i], float(N * M))

        wstS = wst * s1[None, :]
        wntS = wnt * s1[None, :]
        wetS = wet * s1[None, :]
        bS = bvec * s1 + t1
        wntSb = wntS.astype(jnp.bfloat16)
        wetSb = wetS.astype(jnp.bfloat16)
        wF = (wstS[:, :AF], wntSb[:, :AF], wetSb[:, :AF], bS[None, :AF])
        wC = (wstS[:, AF:], wntSb[:, AF:], wetSb[:, AF:], bS[None, AF:])

        ns, acc2 = _pass2(x, gpad, eflat, wF, wC)
        s2, t2 = _bn_affine(acc2, conv_g2[i], conv_b2[i], float(N))
        x = _pass3(x, ns, s2[None, :], t2[None, :])

    pooled = _gather_pool(x, pool_idx)
    out, crys_fea = _final(pooled, W1, b1, Wo, bo)
    return (out, crys_fea)
